# baseline (speedup 1.0000x reference)
import functools

import jax
import jax.numpy as jnp
from jax import lax
from jax.experimental import pallas as pl
from jax.experimental.pallas import tpu as pltpu
from jax.experimental.pallas import tpu_sc as plsc

B = 16384
H = 1024
D_OUT = 128
BM = 512

_NC, _NS = 2, 16
_NW = _NC * _NS
_BPW = B // _NW


def _sc_gather(user_idx, cat0, cat1, user_emb_w, top_cat_emb_w,
               hour_emb_pad):
    mesh = plsc.VectorSubcoreMesh(core_axis_name="c", subcore_axis_name="s")

    @functools.partial(
        pl.kernel,
        mesh=mesh,
        compiler_params=pltpu.CompilerParams(use_tc_tiling_on_sc=False, needs_layout_passes=False),
        out_type=jax.ShapeDtypeStruct((B, 128), jnp.float32),
        scratch_types=[
            pltpu.VMEM((_BPW,), jnp.int32),
            pltpu.VMEM((_BPW,), jnp.int32),
            pltpu.VMEM((_BPW,), jnp.int32),
            pltpu.VMEM((_BPW, 32), jnp.float32),
            pltpu.VMEM((_BPW, 16), jnp.float32),
            pltpu.VMEM((_BPW, 16), jnp.float32),
            pltpu.VMEM((1000 * 16 + 64,), jnp.float32),
            pltpu.SemaphoreType.DMA,
            pltpu.SemaphoreType.DMA,
            pltpu.SemaphoreType.DMA,
        ],
    )
    def k(uidx_hbm, c0_hbm, c1_hbm, ut_hbm, tab_hbm,
          out_x,
          uidx_v, c0_v, c1_v, ue_v, ce_v, he_v, tab_v,
          sem0, sem1, sem2):
        wid = lax.axis_index("s") * _NC + lax.axis_index("c")
        base = wid * _BPW
        i0 = pltpu.async_copy(uidx_hbm.at[pl.ds(base, _BPW)], uidx_v, sem0)
        i1 = pltpu.async_copy(c0_hbm.at[pl.ds(base, _BPW)], c0_v, sem1)
        i2 = pltpu.async_copy(c1_hbm.at[pl.ds(base, _BPW)], c1_v, sem2)
        t0 = pltpu.async_copy(tab_hbm, tab_v, sem1)
        i0.wait()
        cpu = pltpu.async_copy(ut_hbm.at[uidx_v], ue_v, sem0)
        i1.wait()
        i2.wait()
        t0.wait()
        iota = lax.iota(jnp.int32, 16)

        def step(g, _):
            r0 = g * 16
            c0v = c0_v[pl.ds(r0, 16)]
            c1v = c1_v[pl.ds(r0, 16)]
            cg = c0v * 16
            hg = c1v * 16 + 16000
            rows = r0 + iota
            for d in range(16):
                dl = jnp.full((16,), d, jnp.int32)
                valc = plsc.load_gather(tab_v, [cg + d])
                plsc.store_scatter(ce_v, [rows, dl], valc)
                valh = plsc.load_gather(tab_v, [hg + d])
                plsc.store_scatter(he_v, [rows, dl], valh)
            return _

        lax.fori_loop(0, _BPW // 16, step, 0, unroll=2)
        rows = pl.ds(base, _BPW)
        o1 = pltpu.async_copy(ce_v, out_x.at[rows, pl.ds(32, 16)], sem1)
        o2 = pltpu.async_copy(he_v, out_x.at[rows, pl.ds(48, 16)], sem2)
        cpu.wait()
        o0 = pltpu.async_copy(ue_v, out_x.at[rows, pl.ds(0, 32)], sem0)
        o0.wait()
        o1.wait()
        o2.wait()

    tabs = jnp.concatenate(
        [top_cat_emb_w.reshape(-1), hour_emb_pad.reshape(-1)])
    ut2 = jnp.where(user_emb_w == user_emb_w, user_emb_w, jnp.float32(0))
    return k(user_idx, cat0, cat1, ut2, tabs)


def _tc_mlp(x, ud, pf, w1f, b1f, w2t, b2):
    def body(x_ref, ud_ref, pf_ref, w1_ref, b1_ref,
             w2_ref, b2_ref, out_ref):
        z = jnp.zeros((BM, 7), jnp.bfloat16)
        xc = jnp.concatenate(
            [x_ref[:, 0:64].astype(jnp.bfloat16),
             ud_ref[...].astype(jnp.bfloat16),
             pf_ref[...].astype(jnp.bfloat16), z],
            axis=1)
        h = jnp.dot(xc, w1_ref[...], preferred_element_type=jnp.float32)
        h = jnp.maximum(h + b1_ref[...], 0.0).astype(jnp.bfloat16)
        o = jnp.dot(h, w2_ref[...], preferred_element_type=jnp.float32)
        o = o + b2_ref[...]
        ss = jnp.sum(o * o, axis=1, keepdims=True)
        nrm = jnp.maximum(jnp.sqrt(ss), 1e-12)
        out_ref[...] = o / nrm

    grid = (B // BM,)
    return pl.pallas_call(
        body,
        grid=grid,
        in_specs=[
            pl.BlockSpec((BM, 128), lambda i: (i, 0)),
            pl.BlockSpec((BM, 8), lambda i: (i, 0)),
            pl.BlockSpec((BM, 1), lambda i: (i, 0)),
            pl.BlockSpec((80, H), lambda i: (0, 0)),
            pl.BlockSpec((1, H), lambda i: (0, 0)),
            pl.BlockSpec((H, D_OUT), lambda i: (0, 0)),
            pl.BlockSpec((1, D_OUT), lambda i: (0, 0)),
        ],
        out_specs=pl.BlockSpec((BM, D_OUT), lambda i: (i, 0)),
        out_shape=jax.ShapeDtypeStruct((B, D_OUT), jnp.float32),
    )(x, ud, pf, w1f, b1f, w2t, b2)


def kernel(user_idx, user_cat, user_dense, user_emb_w, top_cat_emb_w,
           hour_emb_w, W1, b1, gamma, beta, run_mean, run_var, W2, b2):
    uidx = user_idx.astype(jnp.int32)
    cat0 = user_cat[:, 0].astype(jnp.int32)
    cat1 = user_cat[:, 1].astype(jnp.int32)
    pf = user_cat[:, 3:4].astype(jnp.float32)

    ht_pad = jnp.pad(hour_emb_w, ((0, 0), (0, 8)))

    s = gamma * lax.rsqrt(run_var + 1e-5)
    b1f = ((b1 - run_mean) * s + beta).reshape(1, H)
    w1s = W1 * s[:, None]
    w1t = w1s.T
    w1f = jnp.concatenate(
        [w1t[0:32], w1t[32:48], w1t[48:56], jnp.zeros((8, H), jnp.float32),
         w1t[56:64], w1t[64:65], jnp.zeros((7, H), jnp.float32)], axis=0)
    w1f = w1f.astype(jnp.bfloat16)
    w2t = W2.T.astype(jnp.bfloat16)
    b2r = b2.reshape(1, D_OUT)

    x = _sc_gather(uidx, cat0, cat1, user_emb_w, top_cat_emb_w, ht_pad)
    return _tc_mlp(x, user_dense, pf, w1f, b1f, w2t, b2r)


# speedup vs baseline: 1.3822x; 1.3822x over previous
import functools

import jax
import jax.numpy as jnp
from jax import lax
from jax.experimental import pallas as pl
from jax.experimental.pallas import tpu as pltpu
from jax.experimental.pallas import tpu_sc as plsc

B = 16384
H = 1024
D_OUT = 128
BM = 1024

_NC, _NS = 2, 16
_NW = _NC * _NS
_BPW = B // _NW


def _sc_gather(user_idx, cat0, cat1, user_emb_w, top_cat_emb_w,
               hour_emb_pad):
    mesh = plsc.VectorSubcoreMesh(core_axis_name="c", subcore_axis_name="s")

    @functools.partial(
        pl.kernel,
        mesh=mesh,
        compiler_params=pltpu.CompilerParams(use_tc_tiling_on_sc=False, needs_layout_passes=False),
        out_type=jax.ShapeDtypeStruct((B, 128), jnp.float32),
        scratch_types=[
            pltpu.VMEM((_BPW,), jnp.int32),
            pltpu.VMEM((_BPW,), jnp.int32),
            pltpu.VMEM((_BPW,), jnp.int32),
            pltpu.VMEM((_BPW, 32), jnp.float32),
            pltpu.VMEM((_BPW, 16), jnp.float32),
            pltpu.VMEM((_BPW, 16), jnp.float32),
            pltpu.VMEM((1000 * 16 + 64,), jnp.float32),
            pltpu.SemaphoreType.DMA,
            pltpu.SemaphoreType.DMA,
            pltpu.SemaphoreType.DMA,
        ],
    )
    def k(uidx_hbm, c0_hbm, c1_hbm, ut_hbm, tab_hbm,
          out_x,
          uidx_v, c0_v, c1_v, ue_v, ce_v, he_v, tab_v,
          sem0, sem1, sem2):
        wid = lax.axis_index("s") * _NC + lax.axis_index("c")
        base = wid * _BPW
        i0 = pltpu.async_copy(uidx_hbm.at[pl.ds(base, _BPW)], uidx_v, sem0)
        i1 = pltpu.async_copy(c0_hbm.at[pl.ds(base, _BPW)], c0_v, sem1)
        i2 = pltpu.async_copy(c1_hbm.at[pl.ds(base, _BPW)], c1_v, sem2)
        t0 = pltpu.async_copy(tab_hbm, tab_v, sem1)
        i0.wait()
        cpu = pltpu.async_copy(ut_hbm.at[uidx_v], ue_v, sem0)
        i1.wait()
        i2.wait()
        t0.wait()
        iota = lax.iota(jnp.int32, 16)

        def step(g, _):
            r0 = g * 16
            c0v = c0_v[pl.ds(r0, 16)]
            c1v = c1_v[pl.ds(r0, 16)]
            cg = c0v * 16
            hg = c1v * 16 + 16000
            rows = r0 + iota
            for d in range(16):
                dl = jnp.full((16,), d, jnp.int32)
                valc = plsc.load_gather(tab_v, [cg + d])
                plsc.store_scatter(ce_v, [rows, dl], valc)
                valh = plsc.load_gather(tab_v, [hg + d])
                plsc.store_scatter(he_v, [rows, dl], valh)
            return _

        lax.fori_loop(0, _BPW // 16, step, 0, unroll=2)
        rows = pl.ds(base, _BPW)
        o1 = pltpu.async_copy(ce_v, out_x.at[rows, pl.ds(32, 16)], sem1)
        o2 = pltpu.async_copy(he_v, out_x.at[rows, pl.ds(48, 16)], sem2)
        cpu.wait()
        o0 = pltpu.async_copy(ue_v, out_x.at[rows, pl.ds(0, 32)], sem0)
        o0.wait()
        o1.wait()
        o2.wait()

    tabs = jnp.concatenate(
        [top_cat_emb_w.reshape(-1), hour_emb_pad.reshape(-1)])
    return k(user_idx, cat0, cat1, user_emb_w, tabs)


def _tc_mlp(x, ud, pf, w1f, b1f, w2t, b2):
    def body(x_ref, ud_ref, pf_ref, w1_ref, b1_ref,
             w2_ref, b2_ref, out_ref):
        z = jnp.zeros((BM, 7), jnp.bfloat16)
        xc = jnp.concatenate(
            [x_ref[:, 0:64].astype(jnp.bfloat16),
             ud_ref[...].astype(jnp.bfloat16),
             pf_ref[...].astype(jnp.bfloat16), z],
            axis=1)
        h = jnp.dot(xc, w1_ref[...], preferred_element_type=jnp.float32)
        h = jnp.maximum(h + b1_ref[...], 0.0).astype(jnp.bfloat16)
        o = jnp.dot(h, w2_ref[...], preferred_element_type=jnp.float32)
        o = o + b2_ref[...]
        ss = jnp.sum(o * o, axis=1, keepdims=True)
        nrm = jnp.maximum(jnp.sqrt(ss), 1e-12)
        out_ref[...] = o / nrm

    grid = (B // BM,)
    return pl.pallas_call(
        body,
        grid=grid,
        in_specs=[
            pl.BlockSpec((BM, 128), lambda i: (i, 0)),
            pl.BlockSpec((BM, 8), lambda i: (i, 0)),
            pl.BlockSpec((BM, 1), lambda i: (i, 0)),
            pl.BlockSpec((80, H), lambda i: (0, 0)),
            pl.BlockSpec((1, H), lambda i: (0, 0)),
            pl.BlockSpec((H, D_OUT), lambda i: (0, 0)),
            pl.BlockSpec((1, D_OUT), lambda i: (0, 0)),
        ],
        out_specs=pl.BlockSpec((BM, D_OUT), lambda i: (i, 0)),
        out_shape=jax.ShapeDtypeStruct((B, D_OUT), jnp.float32),
    )(x, ud, pf, w1f, b1f, w2t, b2)


def kernel(user_idx, user_cat, user_dense, user_emb_w, top_cat_emb_w,
           hour_emb_w, W1, b1, gamma, beta, run_mean, run_var, W2, b2):
    uidx = user_idx.astype(jnp.int32)
    cat0 = user_cat[:, 0].astype(jnp.int32)
    cat1 = user_cat[:, 1].astype(jnp.int32)
    pf = user_cat[:, 3:4].astype(jnp.float32)

    ht_pad = jnp.pad(hour_emb_w, ((0, 0), (0, 8)))

    s = gamma * lax.rsqrt(run_var + 1e-5)
    b1f = ((b1 - run_mean) * s + beta).reshape(1, H)
    w1s = W1 * s[:, None]
    w1t = w1s.T
    w1f = jnp.concatenate(
        [w1t[0:32], w1t[32:48], w1t[48:56], jnp.zeros((8, H), jnp.float32),
         w1t[56:64], w1t[64:65], jnp.zeros((7, H), jnp.float32)], axis=0)
    w1f = w1f.astype(jnp.bfloat16)
    w2t = W2.T.astype(jnp.bfloat16)
    b2r = b2.reshape(1, D_OUT)

    x = _sc_gather(uidx, cat0, cat1, user_emb_w, top_cat_emb_w, ht_pad)
    return _tc_mlp(x, user_dense, pf, w1f, b1f, w2t, b2r)


# speedup vs baseline: 1.4313x; 1.0356x over previous
import functools

import jax
import jax.numpy as jnp
from jax import lax
from jax.experimental import pallas as pl
from jax.experimental.pallas import tpu as pltpu
from jax.experimental.pallas import tpu_sc as plsc

B = 16384
H = 1024
D_OUT = 128
BM = 2048

_NC, _NS = 2, 16
_NW = _NC * _NS
_BPW = B // _NW


def _sc_gather(user_idx, cat0, cat1, user_emb_w, top_cat_emb_w,
               hour_emb_pad):
    mesh = plsc.VectorSubcoreMesh(core_axis_name="c", subcore_axis_name="s")

    @functools.partial(
        pl.kernel,
        mesh=mesh,
        compiler_params=pltpu.CompilerParams(use_tc_tiling_on_sc=False, needs_layout_passes=False),
        out_type=jax.ShapeDtypeStruct((B, 128), jnp.float32),
        scratch_types=[
            pltpu.VMEM((_BPW,), jnp.int32),
            pltpu.VMEM((_BPW,), jnp.int32),
            pltpu.VMEM((_BPW,), jnp.int32),
            pltpu.VMEM((_BPW, 32), jnp.float32),
            pltpu.VMEM((_BPW, 16), jnp.float32),
            pltpu.VMEM((_BPW, 16), jnp.float32),
            pltpu.VMEM((1000 * 16 + 64,), jnp.float32),
            pltpu.SemaphoreType.DMA,
            pltpu.SemaphoreType.DMA,
            pltpu.SemaphoreType.DMA,
        ],
    )
    def k(uidx_hbm, c0_hbm, c1_hbm, ut_hbm, tab_hbm,
          out_x,
          uidx_v, c0_v, c1_v, ue_v, ce_v, he_v, tab_v,
          sem0, sem1, sem2):
        wid = lax.axis_index("s") * _NC + lax.axis_index("c")
        base = wid * _BPW
        i0 = pltpu.async_copy(uidx_hbm.at[pl.ds(base, _BPW)], uidx_v, sem0)
        i1 = pltpu.async_copy(c0_hbm.at[pl.ds(base, _BPW)], c0_v, sem1)
        i2 = pltpu.async_copy(c1_hbm.at[pl.ds(base, _BPW)], c1_v, sem2)
        t0 = pltpu.async_copy(tab_hbm, tab_v, sem1)
        i0.wait()
        cpu = pltpu.async_copy(ut_hbm.at[uidx_v], ue_v, sem0)
        i1.wait()
        i2.wait()
        t0.wait()
        iota = lax.iota(jnp.int32, 16)

        def step(g, _):
            r0 = g * 16
            c0v = c0_v[pl.ds(r0, 16)]
            c1v = c1_v[pl.ds(r0, 16)]
            cg = c0v * 16
            hg = c1v * 16 + 16000
            rows = r0 + iota
            for d in range(16):
                dl = jnp.full((16,), d, jnp.int32)
                valc = plsc.load_gather(tab_v, [cg + d])
                plsc.store_scatter(ce_v, [rows, dl], valc)
                valh = plsc.load_gather(tab_v, [hg + d])
                plsc.store_scatter(he_v, [rows, dl], valh)
            return _

        lax.fori_loop(0, _BPW // 16, step, 0, unroll=2)
        rows = pl.ds(base, _BPW)
        o1 = pltpu.async_copy(ce_v, out_x.at[rows, pl.ds(32, 16)], sem1)
        o2 = pltpu.async_copy(he_v, out_x.at[rows, pl.ds(48, 16)], sem2)
        cpu.wait()
        o0 = pltpu.async_copy(ue_v, out_x.at[rows, pl.ds(0, 32)], sem0)
        o0.wait()
        o1.wait()
        o2.wait()

    tabs = jnp.concatenate(
        [top_cat_emb_w.reshape(-1), hour_emb_pad.reshape(-1)])
    return k(user_idx, cat0, cat1, user_emb_w, tabs)


def _tc_mlp(x, ud, pf, w1f, b1f, w2t, b2):
    def body(x_ref, ud_ref, pf_ref, w1_ref, b1_ref,
             w2_ref, b2_ref, out_ref):
        z = jnp.zeros((BM, 7), jnp.bfloat16)
        xc = jnp.concatenate(
            [x_ref[:, 0:64].astype(jnp.bfloat16),
             ud_ref[...].astype(jnp.bfloat16),
             pf_ref[...].astype(jnp.bfloat16), z],
            axis=1)
        h = jnp.dot(xc, w1_ref[...], preferred_element_type=jnp.float32)
        h = jnp.maximum(h + b1_ref[...], 0.0).astype(jnp.bfloat16)
        o = jnp.dot(h, w2_ref[...], preferred_element_type=jnp.float32)
        o = o + b2_ref[...]
        ss = jnp.sum(o * o, axis=1, keepdims=True)
        nrm = jnp.maximum(jnp.sqrt(ss), 1e-12)
        out_ref[...] = o / nrm

    grid = (B // BM,)
    return pl.pallas_call(
        body,
        grid=grid,
        in_specs=[
            pl.BlockSpec((BM, 128), lambda i: (i, 0)),
            pl.BlockSpec((BM, 8), lambda i: (i, 0)),
            pl.BlockSpec((BM, 1), lambda i: (i, 0)),
            pl.BlockSpec((80, H), lambda i: (0, 0)),
            pl.BlockSpec((1, H), lambda i: (0, 0)),
            pl.BlockSpec((H, D_OUT), lambda i: (0, 0)),
            pl.BlockSpec((1, D_OUT), lambda i: (0, 0)),
        ],
        out_specs=pl.BlockSpec((BM, D_OUT), lambda i: (i, 0)),
        out_shape=jax.ShapeDtypeStruct((B, D_OUT), jnp.float32),
    )(x, ud, pf, w1f, b1f, w2t, b2)


def kernel(user_idx, user_cat, user_dense, user_emb_w, top_cat_emb_w,
           hour_emb_w, W1, b1, gamma, beta, run_mean, run_var, W2, b2):
    uidx = user_idx.astype(jnp.int32)
    cat0 = user_cat[:, 0].astype(jnp.int32)
    cat1 = user_cat[:, 1].astype(jnp.int32)
    pf = user_cat[:, 3:4].astype(jnp.float32)

    ht_pad = jnp.pad(hour_emb_w, ((0, 0), (0, 8)))

    s = gamma * lax.rsqrt(run_var + 1e-5)
    b1f = ((b1 - run_mean) * s + beta).reshape(1, H)
    w1s = W1 * s[:, None]
    w1t = w1s.T
    w1f = jnp.concatenate(
        [w1t[0:32], w1t[32:48], w1t[48:56], jnp.zeros((8, H), jnp.float32),
         w1t[56:64], w1t[64:65], jnp.zeros((7, H), jnp.float32)], axis=0)
    w1f = w1f.astype(jnp.bfloat16)
    w2t = W2.T.astype(jnp.bfloat16)
    b2r = b2.reshape(1, D_OUT)

    x = _sc_gather(uidx, cat0, cat1, user_emb_w, top_cat_emb_w, ht_pad)
    return _tc_mlp(x, user_dense, pf, w1f, b1f, w2t, b2r)


# speedup vs baseline: 1.4488x; 1.0122x over previous
import functools

import jax
import jax.numpy as jnp
from jax import lax
from jax.experimental import pallas as pl
from jax.experimental.pallas import tpu as pltpu
from jax.experimental.pallas import tpu_sc as plsc

B = 16384
H = 1024
D_OUT = 128
BM = 4096

_NC, _NS = 2, 16
_NW = _NC * _NS
_BPW = B // _NW


def _sc_gather(user_idx, cat0, cat1, user_emb_w, top_cat_emb_w,
               hour_emb_pad):
    mesh = plsc.VectorSubcoreMesh(core_axis_name="c", subcore_axis_name="s")

    @functools.partial(
        pl.kernel,
        mesh=mesh,
        compiler_params=pltpu.CompilerParams(use_tc_tiling_on_sc=False, needs_layout_passes=False),
        out_type=jax.ShapeDtypeStruct((B, 128), jnp.float32),
        scratch_types=[
            pltpu.VMEM((_BPW,), jnp.int32),
            pltpu.VMEM((_BPW,), jnp.int32),
            pltpu.VMEM((_BPW,), jnp.int32),
            pltpu.VMEM((_BPW, 32), jnp.float32),
            pltpu.VMEM((_BPW, 16), jnp.float32),
            pltpu.VMEM((_BPW, 16), jnp.float32),
            pltpu.VMEM((1000 * 16 + 64,), jnp.float32),
            pltpu.SemaphoreType.DMA,
            pltpu.SemaphoreType.DMA,
            pltpu.SemaphoreType.DMA,
        ],
    )
    def k(uidx_hbm, c0_hbm, c1_hbm, ut_hbm, tab_hbm,
          out_x,
          uidx_v, c0_v, c1_v, ue_v, ce_v, he_v, tab_v,
          sem0, sem1, sem2):
        wid = lax.axis_index("s") * _NC + lax.axis_index("c")
        base = wid * _BPW
        i0 = pltpu.async_copy(uidx_hbm.at[pl.ds(base, _BPW)], uidx_v, sem0)
        i1 = pltpu.async_copy(c0_hbm.at[pl.ds(base, _BPW)], c0_v, sem1)
        i2 = pltpu.async_copy(c1_hbm.at[pl.ds(base, _BPW)], c1_v, sem2)
        t0 = pltpu.async_copy(tab_hbm, tab_v, sem1)
        i0.wait()
        cpu = pltpu.async_copy(ut_hbm.at[uidx_v], ue_v, sem0)
        i1.wait()
        i2.wait()
        t0.wait()
        iota = lax.iota(jnp.int32, 16)

        def step(g, _):
            r0 = g * 16
            c0v = c0_v[pl.ds(r0, 16)]
            c1v = c1_v[pl.ds(r0, 16)]
            cg = c0v * 16
            hg = c1v * 16 + 16000
            rows = r0 + iota
            for d in range(16):
                dl = jnp.full((16,), d, jnp.int32)
                valc = plsc.load_gather(tab_v, [cg + d])
                plsc.store_scatter(ce_v, [rows, dl], valc)
                valh = plsc.load_gather(tab_v, [hg + d])
                plsc.store_scatter(he_v, [rows, dl], valh)
            return _

        lax.fori_loop(0, _BPW // 16, step, 0, unroll=2)
        rows = pl.ds(base, _BPW)
        o1 = pltpu.async_copy(ce_v, out_x.at[rows, pl.ds(32, 16)], sem1)
        o2 = pltpu.async_copy(he_v, out_x.at[rows, pl.ds(48, 16)], sem2)
        cpu.wait()
        o0 = pltpu.async_copy(ue_v, out_x.at[rows, pl.ds(0, 32)], sem0)
        o0.wait()
        o1.wait()
        o2.wait()

    tabs = jnp.concatenate(
        [top_cat_emb_w.reshape(-1), hour_emb_pad.reshape(-1)])
    return k(user_idx, cat0, cat1, user_emb_w, tabs)


def _tc_mlp(x, ud, pf, w1f, b1f, w2t, b2):
    def body(x_ref, ud_ref, pf_ref, w1_ref, b1_ref,
             w2_ref, b2_ref, out_ref):
        z = jnp.zeros((BM, 7), jnp.bfloat16)
        xc = jnp.concatenate(
            [x_ref[:, 0:64].astype(jnp.bfloat16),
             ud_ref[...].astype(jnp.bfloat16),
             pf_ref[...].astype(jnp.bfloat16), z],
            axis=1)
        h = jnp.dot(xc, w1_ref[...], preferred_element_type=jnp.float32)
        h = jnp.maximum(h + b1_ref[...], 0.0).astype(jnp.bfloat16)
        o = jnp.dot(h, w2_ref[...], preferred_element_type=jnp.float32)
        o = o + b2_ref[...]
        ss = jnp.sum(o * o, axis=1, keepdims=True)
        nrm = jnp.maximum(jnp.sqrt(ss), 1e-12)
        out_ref[...] = o / nrm

    grid = (B // BM,)
    return pl.pallas_call(
        body,
        grid=grid,
        in_specs=[
            pl.BlockSpec((BM, 128), lambda i: (i, 0)),
            pl.BlockSpec((BM, 8), lambda i: (i, 0)),
            pl.BlockSpec((BM, 1), lambda i: (i, 0)),
            pl.BlockSpec((80, H), lambda i: (0, 0)),
            pl.BlockSpec((1, H), lambda i: (0, 0)),
            pl.BlockSpec((H, D_OUT), lambda i: (0, 0)),
            pl.BlockSpec((1, D_OUT), lambda i: (0, 0)),
        ],
        out_specs=pl.BlockSpec((BM, D_OUT), lambda i: (i, 0)),
        out_shape=jax.ShapeDtypeStruct((B, D_OUT), jnp.float32),
    )(x, ud, pf, w1f, b1f, w2t, b2)


def kernel(user_idx, user_cat, user_dense, user_emb_w, top_cat_emb_w,
           hour_emb_w, W1, b1, gamma, beta, run_mean, run_var, W2, b2):
    uidx = user_idx.astype(jnp.int32)
    cat0 = user_cat[:, 0].astype(jnp.int32)
    cat1 = user_cat[:, 1].astype(jnp.int32)
    pf = user_cat[:, 3:4].astype(jnp.float32)

    ht_pad = jnp.pad(hour_emb_w, ((0, 0), (0, 8)))

    s = gamma * lax.rsqrt(run_var + 1e-5)
    b1f = ((b1 - run_mean) * s + beta).reshape(1, H)
    w1s = W1 * s[:, None]
    w1t = w1s.T
    w1f = jnp.concatenate(
        [w1t[0:32], w1t[32:48], w1t[48:56], jnp.zeros((8, H), jnp.float32),
         w1t[56:64], w1t[64:65], jnp.zeros((7, H), jnp.float32)], axis=0)
    w1f = w1f.astype(jnp.bfloat16)
    w2t = W2.T.astype(jnp.bfloat16)
    b2r = b2.reshape(1, D_OUT)

    x = _sc_gather(uidx, cat0, cat1, user_emb_w, top_cat_emb_w, ht_pad)
    return _tc_mlp(x, user_dense, pf, w1f, b1f, w2t, b2r)
